# baseline (speedup 1.0000x reference)
import functools

import jax
import jax.numpy as jnp
from jax import lax
from jax.experimental import pallas as pl
from jax.experimental.pallas import tpu as pltpu
from jax.experimental.pallas import tpu_sc as plsc

NC = 2
NS = 16
NW = NC * NS
L = 16

BATCH = 4096
HIST = 200
HIDDEN = 64
B = BATCH * HIST
C = 128
ROWS_PER_W = B // NW
NCHUNK = ROWS_PER_W // C
NBUF = 2


_GATHER_DNUMS = lax.GatherDimensionNumbers(
    offset_dims=(), collapsed_slice_dims=(0,), start_index_map=(0,)
)


def _lane_perm(v, perm):
    return lax.gather(
        v,
        perm[:, None],
        _GATHER_DNUMS,
        slice_sizes=(1,),
        mode=lax.GatherScatterMode.PROMISE_IN_BOUNDS,
    )


def _lane_sum(v):
    lanes = lax.iota(jnp.int32, L)
    for d in (8, 4, 2, 1):
        v = v + _lane_perm(v, lanes ^ d)
    return v


def _rsqrt_vec(s):
    i = lax.bitcast_convert_type(s, jnp.int32)
    i = jnp.int32(0x5F3759DF) - (i >> 1)
    y = lax.bitcast_convert_type(i, jnp.float32)
    for _ in range(2):
        y = y * (1.5 - 0.5 * s * y * y)
    return y


def _norm_row(inb, outb, b, r):
    v0 = inb[b, r, pl.ds(0, L)]
    v1 = inb[b, r, pl.ds(L, L)]
    v2 = inb[b, r, pl.ds(2 * L, L)]
    v3 = inb[b, r, pl.ds(3 * L, L)]
    ss = _lane_sum(v0 * v0 + v1 * v1 + v2 * v2 + v3 * v3)
    sc = _rsqrt_vec(ss)
    outb[b, r, pl.ds(0, L)] = v0 * sc
    outb[b, r, pl.ds(L, L)] = v1 * sc
    outb[b, r, pl.ds(2 * L, L)] = v2 * sc
    outb[b, r, pl.ds(3 * L, L)] = v3 * sc


@functools.partial(
    pl.kernel,
    out_type=jax.ShapeDtypeStruct((B, HIDDEN), jnp.float32),
    mesh=plsc.VectorSubcoreMesh(
        core_axis_name="c", subcore_axis_name="s", num_cores=NC
    ),
    compiler_params=pltpu.CompilerParams(use_tc_tiling_on_sc=False),
    scratch_types=[
        pltpu.VMEM((NCHUNK, C), jnp.int32),
        pltpu.VMEM((NBUF, C, HIDDEN), jnp.float32),
        pltpu.VMEM((NBUF, C, HIDDEN), jnp.float32),
        pltpu.SemaphoreType.DMA((NBUF,)),
        pltpu.SemaphoreType.DMA((NBUF,)),
    ],
)
def _emb_norm(idx_hbm, table_hbm, out_hbm, idx_v, inb, outb, gsem, osem):
    wid = lax.axis_index("s") * NC + lax.axis_index("c")
    base = wid * ROWS_PER_W

    pltpu.sync_copy(idx_hbm.at[wid], idx_v)

    def _gather(j, b):
        return pltpu.make_async_copy(
            table_hbm.at[idx_v.at[j]], inb.at[b], gsem.at[b]
        )

    def _put(j, b):
        return pltpu.make_async_copy(
            outb.at[b], out_hbm.at[pl.ds(base + j * C, C)], osem.at[b]
        )

    for b in range(NBUF):
        _gather(b, b).start()

    def chunk_body(i, carry):
        for b in range(NBUF):
            j = i * NBUF + b
            _gather(j, b).wait()

            @pl.when(j >= NBUF)
            def _():
                _put(j - NBUF, b).wait()

            def rows_body(g, c):
                r0 = g * 8
                for rr in range(8):
                    _norm_row(inb, outb, b, r0 + rr)
                return c

            lax.fori_loop(0, C // 8, rows_body, 0)

            _put(j, b).start()

            @pl.when(j + NBUF < NCHUNK)
            def _():
                _gather(j + NBUF, b).start()
        return carry

    lax.fori_loop(0, NCHUNK // NBUF, chunk_body, 0)

    for b in range(NBUF):
        _put(0, b).wait()


def kernel(inputs, embedding_weight):
    idx = inputs.reshape(NW, NCHUNK, C).astype(jnp.int32)
    out = _emb_norm(idx, embedding_weight)
    return out.reshape(BATCH, HIST, HIDDEN)
